# Initial kernel scaffold; baseline (speedup 1.0000x reference)
import jax
import jax.numpy as jnp
from jax.experimental import pallas as pl


def kernel(x, W0, al0, ar0, b0, W1, al1, ar1, b1, W2, al2, ar2, b2, ei0, ei1, ei2):
    raise NotImplementedError("write your pallas kernel here")


# speedup vs baseline: 10.9543x; 10.9543x over previous
import jax
import jax.numpy as jnp
from jax import lax
from jax.experimental import pallas as pl
from jax.experimental.pallas import tpu as pltpu
from jax.experimental.pallas import tpu_sc as plsc

N = 10000
NP = 10240
E = 160000
IN = 256
H = 4
OUT = 256
R = 3
CW = 64
C8 = 16
BN = 400
NB = N // BN
ECH = 2000
NCH = E // ECH
SG = 80
NSG = ECH // SG
NC, NS, L = 2, 16, 16

_i32 = jnp.int32
_f32 = jnp.float32


def _splat(v):
    return jnp.full((L,), v, _i32)


def _iota16():
    return lax.broadcasted_iota(_i32, (L,), 0)


def _sc_mesh():
    return plsc.VectorSubcoreMesh(core_axis_name="c", subcore_axis_name="s",
                                  num_cores=NC, num_subcores=NS)


def _phase_a_body(x_ref, w_ref, al_ref, ar_ref, z_ref, el_ref, er_ref):
    z = jnp.dot(x_ref[...], w_ref[0], preferred_element_type=_f32)
    zr = z.reshape(BN, H, OUT)
    el_ref[0] = jnp.sum(zr * al_ref[0][None], axis=-1)
    er_ref[0] = jnp.sum(zr * ar_ref[0][None], axis=-1)
    for c in range(C8):
        z_ref[0, c] = z[:, c * CW:(c + 1) * CW]


def _phase_a(x, Ws, als, ars):
    return pl.pallas_call(
        _phase_a_body,
        grid=(R, NB),
        in_specs=[
            pl.BlockSpec((BN, IN), lambda r, i: (i, 0)),
            pl.BlockSpec((1, IN, H * OUT), lambda r, i: (r, 0, 0)),
            pl.BlockSpec((1, H, OUT), lambda r, i: (r, 0, 0)),
            pl.BlockSpec((1, H, OUT), lambda r, i: (r, 0, 0)),
        ],
        out_specs=[
            pl.BlockSpec((1, C8, BN, CW), lambda r, i: (r, 0, i, 0)),
            pl.BlockSpec((1, BN, H), lambda r, i: (r, i, 0)),
            pl.BlockSpec((1, BN, H), lambda r, i: (r, i, 0)),
        ],
        out_shape=[
            jax.ShapeDtypeStruct((R, C8, N, CW), _f32),
            jax.ShapeDtypeStruct((R, N, H), _f32),
            jax.ShapeDtypeStruct((R, N, H), _f32),
        ],
        compiler_params=pltpu.CompilerParams(
            dimension_semantics=("arbitrary", "arbitrary")),
    )(x, Ws, als, ars)


def _phase_a2_body(el_ref, er_ref, elt_ref, ert_ref):
    elt_ref[0] = el_ref[0].T
    ert_ref[0] = er_ref[0].T


def _phase_a2(eln, ern):
    return pl.pallas_call(
        _phase_a2_body,
        grid=(R,),
        in_specs=[
            pl.BlockSpec((1, N, H), lambda r: (r, 0, 0)),
            pl.BlockSpec((1, N, H), lambda r: (r, 0, 0)),
        ],
        out_specs=[
            pl.BlockSpec((1, H, N), lambda r: (r, 0, 0)),
            pl.BlockSpec((1, H, N), lambda r: (r, 0, 0)),
        ],
        out_shape=[
            jax.ShapeDtypeStruct((R, H, N), _f32),
            jax.ShapeDtypeStruct((R, H, N), _f32),
        ],
    )(eln, ern)


def _phase_b_body(elt, ert, srcs, d2, denp, ext,
                  elbuf, erbuf, sbuf, d2buf, exb0, exb1, exb2, exb3,
                  rowbuf, zbuf, mbuf, dacc):
    core = lax.axis_index("c")
    sub = lax.axis_index("s")
    iota = _iota16()

    @pl.loop(0, NP // NS, unroll=8)
    def _(i):
        zbuf[i, :] = jnp.zeros((L,), _f32)

    pltpu.sync_copy(zbuf, dacc.at[pl.ds(sub * (NP // NS), NP // NS)])
    plsc.subcore_barrier()

    @pl.when(sub < 12)
    def _work():
        w = sub * NC + core
        r = w // 8
        eighth = w % 8
        pltpu.sync_copy(elt.at[r], elbuf)
        pltpu.sync_copy(ert.at[r], erbuf)

        @pl.loop(0, SG)
        def _(g):
            rowbuf[g, :] = jnp.zeros((L,), _f32)

        iota16 = _iota16()

        def _allmax(m):
            for s in (1, 2, 4, 8):
                mbuf[pl.ds(0, L)] = m
                v = plsc.load_gather(
                    mbuf, [jnp.bitwise_xor(iota16, _splat(s))])
                m = jnp.maximum(m, v)
            return m

        cvec = []
        for h in range(H):
            def _mx(buf):
                def body(g, m):
                    return jnp.maximum(m, buf[h, pl.ds(g * L, L)])
                m = lax.fori_loop(0, N // L, body,
                                  jnp.full((L,), -3.4e38, _f32))
                return _allmax(m)
            ch = _mx(elbuf) + _mx(erbuf)
            cvec.append(jnp.maximum(ch, 0.2 * ch))

        @pl.loop(0, 10)
        def _chunk(k):
            cidx = eighth * 10 + k
            pltpu.sync_copy(srcs.at[pl.ds(r * E + cidx * ECH, ECH)], sbuf)
            pltpu.sync_copy(d2.at[r, cidx], d2buf)

            @pl.loop(0, ECH // L, unroll=2)
            def _g16(g):
                s = sbuf[pl.ds(g * L, L)]
                d = d2buf[g // 5, pl.ds((g % 5) * L, L)]
                exbufs = [exb0, exb1, exb2, exb3]
                for h in range(H):
                    elg = plsc.load_gather(elbuf, [_splat(h), s])
                    erg = plsc.load_gather(erbuf, [_splat(h), d])
                    t = elg + erg
                    e = jnp.maximum(t, 0.2 * t)
                    exbufs[h][pl.ds(g * L, L)] = jnp.exp(e - cvec[h])

            for h, exb in enumerate((exb0, exb1, exb2, exb3)):
                pltpu.sync_copy(
                    exb, ext.at[pl.ds((r * H + h) * E + cidx * ECH, ECH)])

            @pl.loop(0, NSG)
            def _sg(sg):
                for h, exb in enumerate((exb0, exb1, exb2, exb3)):
                    for t in range(SG // L):
                        exv = exb[pl.ds(sg * SG + t * L, L)]
                        gi = iota + t * L
                        ci = _splat(r * H + h)
                        plsc.store_scatter(rowbuf, [gi, ci], exv)
                pltpu.sync_copy(rowbuf, dacc.at[d2buf.at[sg]], add=True)

    plsc.subcore_barrier()

    @pl.when(sub == 0)
    def _out():
        pltpu.sync_copy(dacc, denp.at[core])


def _phase_b(elt, ert, srcs, d2):
    return pl.kernel(
        _phase_b_body,
        out_type=[
            jax.ShapeDtypeStruct((NC, NP, 16), _f32),
            jax.ShapeDtypeStruct((R * H * E,), _f32),
        ],
        mesh=_sc_mesh(),
        compiler_params=pltpu.CompilerParams(needs_layout_passes=False,
                                             use_tc_tiling_on_sc=False),
        scratch_types=[
            pltpu.VMEM((H, N), _f32),
            pltpu.VMEM((H, N), _f32),
            pltpu.VMEM((ECH,), _i32),
            pltpu.VMEM((NSG, SG), _i32),
            pltpu.VMEM((ECH,), _f32),
            pltpu.VMEM((ECH,), _f32),
            pltpu.VMEM((ECH,), _f32),
            pltpu.VMEM((ECH,), _f32),
            pltpu.VMEM((SG, 16), _f32),
            pltpu.VMEM((NP // NS, 16), _f32),
            pltpu.VMEM((128,), _f32),
            pltpu.VMEM_SHARED((NP, 16), _f32),
        ],
    )(elt, ert, srcs, d2)


def _phase_b2_body(denp_ref, rden_ref):
    d = denp_ref[0] + denp_ref[1]
    rd = 1.0 / jnp.maximum(d, 1e-30)
    rden_ref[...] = rd.T.reshape(16 * NP)


def _phase_b2(denp):
    return pl.pallas_call(
        _phase_b2_body,
        grid=(1,),
        in_specs=[pl.BlockSpec((NC, NP, 16), lambda i: (0, 0, 0))],
        out_specs=pl.BlockSpec((16 * NP,), lambda i: (0,)),
        out_shape=jax.ShapeDtypeStruct((16 * NP,), _f32),
    )(denp)


def _phase_c_body(zflat, ext, rden, srcs, d2, outacc,
                  rdenbuf, exsl, albuf, gidx, sbuf, d2buf, rows,
                  zbuf, sem, acc):
    core = lax.axis_index("c")
    sub = lax.axis_index("s")
    et = E // NS

    @pl.loop(0, 128, unroll=8)
    def _(i):
        for kk in range(CW // L):
            zbuf[i, pl.ds(kk * L, L)] = jnp.zeros((L,), _f32)

    @pl.loop(0, C8 // NC)
    def _chunk(cl):
        cg = core * (C8 // NC) + cl
        h = cg // (C8 // H)
        for j in range(5):
            pltpu.sync_copy(zbuf, acc.at[pl.ds(sub * 640 + j * 128, 128)])
        plsc.subcore_barrier()

        @pl.loop(0, R)
        def _rel(r):
            rh = r * H + h
            pltpu.sync_copy(rden.at[pl.ds(rh * NP, NP)], rdenbuf)
            pltpu.sync_copy(ext.at[pl.ds((r * H + h) * E + sub * et, et)],
                            exsl)
            off = (r * C8 + cg) * N
            for k in range(5):
                ci = sub * 5 + k
                pltpu.sync_copy(srcs.at[pl.ds(r * E + ci * ECH, ECH)], sbuf)
                pltpu.sync_copy(d2.at[r, ci], d2buf.at[pl.ds(k * NSG, NSG)])

                @pl.loop(0, ECH // L, unroll=8)
                def _off(g):
                    gidx[pl.ds(k * ECH + g * L, L)] = (
                        sbuf[pl.ds(g * L, L)] + off)

            @pl.loop(0, et // L, unroll=8)
            def _alpha(g):
                d = d2buf[g // 5, pl.ds((g % 5) * L, L)]
                rd = plsc.load_gather(rdenbuf, [d])
                albuf[pl.ds(g * L, L)] = exsl[pl.ds(g * L, L)] * rd

            @pl.loop(0, et // SG)
            def _sg(sg):
                pltpu.async_copy(zflat.at[gidx.at[pl.ds(sg * SG, SG)]],
                                 rows, sem).wait()

                @pl.loop(0, SG, unroll=2)
                def _scale(g):
                    a = plsc.load_gather(albuf,
                                         [jnp.full((L,), sg * SG + g, _i32)])
                    for kk in range(CW // L):
                        rows[g, pl.ds(kk * L, L)] = (
                            rows[g, pl.ds(kk * L, L)] * a)

                pltpu.sync_copy(rows, acc.at[d2buf.at[sg]], add=True)

        plsc.subcore_barrier()
        pltpu.sync_copy(acc.at[pl.ds(sub * 640, 640)],
                        outacc.at[cg, pl.ds(sub * 640, 640)])
        plsc.subcore_barrier()


def _phase_c(zflat, ext, rden, srcs, d2):
    return pl.kernel(
        _phase_c_body,
        out_type=jax.ShapeDtypeStruct((C8, NP, CW), _f32),
        mesh=_sc_mesh(),
        compiler_params=pltpu.CompilerParams(needs_layout_passes=False,
                                             use_tc_tiling_on_sc=False),
        scratch_types=[
            pltpu.VMEM((NP,), _f32),
            pltpu.VMEM((E // NS,), _f32),
            pltpu.VMEM((E // NS,), _f32),
            pltpu.VMEM((E // NS,), _i32),
            pltpu.VMEM((ECH,), _i32),
            pltpu.VMEM((E // NS // SG, SG), _i32),
            pltpu.VMEM((SG, CW), _f32),
            pltpu.VMEM((128, CW), _f32),
            pltpu.SemaphoreType.DMA,
            pltpu.VMEM_SHARED((NP, CW), _f32),
        ],
    )(zflat, ext, rden, srcs, d2)


def _phase_d_body(acc_ref, bs_ref, out_ref):
    bsum = jnp.sum(bs_ref[...], axis=0)
    nq = OUT // CW
    for h in range(H):
        for q in range(nq):
            out_ref[:, h, q * CW:(q + 1) * CW] = (
                acc_ref[h * nq + q]
                + bsum[h, q * CW:(q + 1) * CW][None])


def _phase_d(outacc, bs):
    return pl.pallas_call(
        _phase_d_body,
        grid=(NB,),
        in_specs=[
            pl.BlockSpec((C8, BN, CW), lambda i: (0, i, 0)),
            pl.BlockSpec((R, H, OUT), lambda i: (0, 0, 0)),
        ],
        out_specs=pl.BlockSpec((BN, H, OUT), lambda i: (i, 0, 0)),
        out_shape=jax.ShapeDtypeStruct((N, H, OUT), _f32),
    )(outacc, bs)


def kernel(x, W0, al0, ar0, b0, W1, al1, ar1, b1, W2, al2, ar2, b2,
           ei0, ei1, ei2):
    Ws = jnp.stack([W0, W1, W2])
    als = jnp.stack([al0, al1, al2]).reshape(R, H, OUT)
    ars = jnp.stack([ar0, ar1, ar2]).reshape(R, H, OUT)
    bs = jnp.stack([b0, b1, b2])
    srcs = jnp.stack([ei0[0], ei1[0], ei2[0]]).reshape(R * E)
    d2 = jnp.stack([ei0[1], ei1[1], ei2[1]]).reshape(R, NCH, NSG, SG)

    Z, eln, ern = _phase_a(x, Ws, als, ars)
    elt, ert = _phase_a2(eln, ern)
    zflat = Z.reshape(R * C8 * N, CW)
    denp, ext = _phase_b(elt, ert, srcs, d2)
    rden = _phase_b2(denp)
    outacc = _phase_c(zflat, ext, rden, srcs, d2)
    return _phase_d(outacc, bs)


# speedup vs baseline: 18.0234x; 1.6453x over previous
import jax
import jax.numpy as jnp
from jax import lax
from jax.experimental import pallas as pl
from jax.experimental.pallas import tpu as pltpu
from jax.experimental.pallas import tpu_sc as plsc

N = 10000
NP = 10240
E = 160000
IN = 256
H = 4
OUT = 256
R = 3
CW = 64
C8 = 16
BN = 400
NB = N // BN
ECH = 2000
NCH = E // ECH
SG = 80
NSG = ECH // SG
NC, NS, L = 2, 16, 16

_i32 = jnp.int32
_f32 = jnp.float32


def _splat(v):
    return jnp.full((L,), v, _i32)


def _iota16():
    return lax.broadcasted_iota(_i32, (L,), 0)


def _sc_mesh():
    return plsc.VectorSubcoreMesh(core_axis_name="c", subcore_axis_name="s",
                                  num_cores=NC, num_subcores=NS)


def _phase_a_body(x_ref, w_ref, al_ref, ar_ref, z_ref, el_ref, er_ref):
    z = jnp.dot(x_ref[...], w_ref[0], preferred_element_type=_f32)
    zr = z.reshape(BN, H, OUT)
    el_ref[0] = jnp.sum(zr * al_ref[0][None], axis=-1)
    er_ref[0] = jnp.sum(zr * ar_ref[0][None], axis=-1)
    for c in range(C8):
        z_ref[0, c] = z[:, c * CW:(c + 1) * CW]


def _phase_a(x, Ws, als, ars):
    return pl.pallas_call(
        _phase_a_body,
        grid=(R, NB),
        in_specs=[
            pl.BlockSpec((BN, IN), lambda r, i: (i, 0)),
            pl.BlockSpec((1, IN, H * OUT), lambda r, i: (r, 0, 0)),
            pl.BlockSpec((1, H, OUT), lambda r, i: (r, 0, 0)),
            pl.BlockSpec((1, H, OUT), lambda r, i: (r, 0, 0)),
        ],
        out_specs=[
            pl.BlockSpec((1, C8, BN, CW), lambda r, i: (r, 0, i, 0)),
            pl.BlockSpec((1, BN, H), lambda r, i: (r, i, 0)),
            pl.BlockSpec((1, BN, H), lambda r, i: (r, i, 0)),
        ],
        out_shape=[
            jax.ShapeDtypeStruct((R, C8, N, CW), _f32),
            jax.ShapeDtypeStruct((R, N, H), _f32),
            jax.ShapeDtypeStruct((R, N, H), _f32),
        ],
        compiler_params=pltpu.CompilerParams(
            dimension_semantics=("arbitrary", "arbitrary")),
    )(x, Ws, als, ars)


def _phase_a2_body(el_ref, er_ref, elt_ref, ert_ref):
    elt_ref[0] = el_ref[0].T
    ert_ref[0] = er_ref[0].T


def _phase_a2(eln, ern):
    return pl.pallas_call(
        _phase_a2_body,
        grid=(R,),
        in_specs=[
            pl.BlockSpec((1, N, H), lambda r: (r, 0, 0)),
            pl.BlockSpec((1, N, H), lambda r: (r, 0, 0)),
        ],
        out_specs=[
            pl.BlockSpec((1, H, N), lambda r: (r, 0, 0)),
            pl.BlockSpec((1, H, N), lambda r: (r, 0, 0)),
        ],
        out_shape=[
            jax.ShapeDtypeStruct((R, H, N), _f32),
            jax.ShapeDtypeStruct((R, H, N), _f32),
        ],
    )(eln, ern)


def _phase_b_body(elt, ert, srcs, d2, denp, ext,
                  elbuf, erbuf, sbuf, d2buf, exb0, exb1, exb2, exb3,
                  rowbuf, zbuf, mbuf, dacc):
    core = lax.axis_index("c")
    sub = lax.axis_index("s")
    iota = _iota16()

    @pl.loop(0, NP // NS, unroll=8)
    def _(i):
        zbuf[i, :] = jnp.zeros((L,), _f32)

    pltpu.sync_copy(zbuf, dacc.at[pl.ds(sub * (NP // NS), NP // NS)])
    plsc.subcore_barrier()

    @pl.when(sub < 12)
    def _work():
        w = sub * NC + core
        r = w // 8
        eighth = w % 8
        pltpu.sync_copy(elt.at[r], elbuf)
        pltpu.sync_copy(ert.at[r], erbuf)

        @pl.loop(0, SG)
        def _(g):
            rowbuf[g, :] = jnp.zeros((L,), _f32)

        iota16 = _iota16()

        def _allmax(m):
            for s in (1, 2, 4, 8):
                mbuf[pl.ds(0, L)] = m
                v = plsc.load_gather(
                    mbuf, [jnp.bitwise_xor(iota16, _splat(s))])
                m = jnp.maximum(m, v)
            return m

        cvec = []
        for h in range(H):
            def _mx(buf):
                def body(g, m):
                    return jnp.maximum(m, buf[h, pl.ds(g * L, L)])
                m = lax.fori_loop(0, N // L, body,
                                  jnp.full((L,), -3.4e38, _f32))
                return _allmax(m)
            ch = _mx(elbuf) + _mx(erbuf)
            cvec.append(jnp.maximum(ch, 0.2 * ch))

        @pl.loop(0, 10)
        def _chunk(k):
            cidx = eighth * 10 + k
            pltpu.sync_copy(srcs.at[pl.ds(r * E + cidx * ECH, ECH)], sbuf)
            pltpu.sync_copy(d2.at[r, cidx], d2buf)

            @pl.loop(0, ECH // L, unroll=2)
            def _g16(g):
                s = sbuf[pl.ds(g * L, L)]
                d = d2buf[g // 5, pl.ds((g % 5) * L, L)]
                exbufs = [exb0, exb1, exb2, exb3]
                for h in range(H):
                    elg = plsc.load_gather(elbuf, [_splat(h), s])
                    erg = plsc.load_gather(erbuf, [_splat(h), d])
                    t = elg + erg
                    e = jnp.maximum(t, 0.2 * t)
                    exbufs[h][pl.ds(g * L, L)] = jnp.exp(e - cvec[h])

            for h, exb in enumerate((exb0, exb1, exb2, exb3)):
                pltpu.sync_copy(
                    exb, ext.at[pl.ds((r * H + h) * E + cidx * ECH, ECH)])

            @pl.loop(0, NSG)
            def _sg(sg):
                for h, exb in enumerate((exb0, exb1, exb2, exb3)):
                    for t in range(SG // L):
                        exv = exb[pl.ds(sg * SG + t * L, L)]
                        gi = iota + t * L
                        ci = _splat(r * H + h)
                        plsc.store_scatter(rowbuf, [gi, ci], exv)
                pltpu.sync_copy(rowbuf, dacc.at[d2buf.at[sg]], add=True)

    plsc.subcore_barrier()

    @pl.when(sub == 0)
    def _out():
        pltpu.sync_copy(dacc, denp.at[core])


def _phase_b(elt, ert, srcs, d2):
    return pl.kernel(
        _phase_b_body,
        out_type=[
            jax.ShapeDtypeStruct((NC, NP, 16), _f32),
            jax.ShapeDtypeStruct((R * H * E,), _f32),
        ],
        mesh=_sc_mesh(),
        compiler_params=pltpu.CompilerParams(needs_layout_passes=False,
                                             use_tc_tiling_on_sc=False),
        scratch_types=[
            pltpu.VMEM((H, N), _f32),
            pltpu.VMEM((H, N), _f32),
            pltpu.VMEM((ECH,), _i32),
            pltpu.VMEM((NSG, SG), _i32),
            pltpu.VMEM((ECH,), _f32),
            pltpu.VMEM((ECH,), _f32),
            pltpu.VMEM((ECH,), _f32),
            pltpu.VMEM((ECH,), _f32),
            pltpu.VMEM((SG, 16), _f32),
            pltpu.VMEM((NP // NS, 16), _f32),
            pltpu.VMEM((128,), _f32),
            pltpu.VMEM_SHARED((NP, 16), _f32),
        ],
    )(elt, ert, srcs, d2)


def _phase_b2_body(denp_ref, rden_ref):
    d = denp_ref[0] + denp_ref[1]
    rd = 1.0 / jnp.maximum(d, 1e-30)
    rden_ref[...] = rd.T.reshape(16 * NP)


def _phase_b2(denp):
    return pl.pallas_call(
        _phase_b2_body,
        grid=(1,),
        in_specs=[pl.BlockSpec((NC, NP, 16), lambda i: (0, 0, 0))],
        out_specs=pl.BlockSpec((16 * NP,), lambda i: (0,)),
        out_shape=jax.ShapeDtypeStruct((16 * NP,), _f32),
    )(denp)


def _phase_c_body(zflat, ext, rden, srcs, d2, outacc,
                  rdenbuf, exsl, albuf, gidx, sbuf, d2buf, rows, rows1,
                  zbuf, sem, sem1, acc):
    core = lax.axis_index("c")
    sub = lax.axis_index("s")
    et = E // NS

    @pl.loop(0, 128, unroll=8)
    def _(i):
        for kk in range(CW // L):
            zbuf[i, pl.ds(kk * L, L)] = jnp.zeros((L,), _f32)

    @pl.loop(0, C8 // NC)
    def _chunk(cl):
        cg = core * (C8 // NC) + cl
        h = cg // (C8 // H)
        for j in range(5):
            pltpu.sync_copy(zbuf, acc.at[pl.ds(sub * 640 + j * 128, 128)])
        plsc.subcore_barrier()

        @pl.loop(0, R)
        def _rel(r):
            rh = r * H + h
            pltpu.sync_copy(rden.at[pl.ds(rh * NP, NP)], rdenbuf)
            pltpu.sync_copy(ext.at[pl.ds((r * H + h) * E + sub * et, et)],
                            exsl)
            off = (r * C8 + cg) * N
            for k in range(5):
                ci = sub * 5 + k
                pltpu.sync_copy(srcs.at[pl.ds(r * E + ci * ECH, ECH)], sbuf)
                pltpu.sync_copy(d2.at[r, ci], d2buf.at[pl.ds(k * NSG, NSG)])

                @pl.loop(0, ECH // L, unroll=8)
                def _off(g):
                    gidx[pl.ds(k * ECH + g * L, L)] = (
                        sbuf[pl.ds(g * L, L)] + off)

            @pl.loop(0, et // L, unroll=8)
            def _alpha(g):
                d = d2buf[g // 5, pl.ds((g % 5) * L, L)]
                rd = plsc.load_gather(rdenbuf, [d])
                albuf[pl.ds(g * L, L)] = exsl[pl.ds(g * L, L)] * rd

            def _gather(sg, buf, sm):
                return pltpu.async_copy(
                    zflat.at[gidx.at[pl.ds(sg * SG, SG)]], buf, sm)

            def _wait(buf, sm):
                pltpu.make_async_copy(
                    zflat.at[gidx.at[pl.ds(0, SG)]], buf, sm).wait()

            def _do(sg, buf):
                @pl.loop(0, SG, unroll=4)
                def _scale(g):
                    a = plsc.load_gather(albuf,
                                         [jnp.full((L,), sg * SG + g, _i32)])
                    for kk in range(CW // L):
                        buf[g, pl.ds(kk * L, L)] = (
                            buf[g, pl.ds(kk * L, L)] * a)

                pltpu.sync_copy(buf, acc.at[d2buf.at[sg]], add=True)

            _gather(0, rows, sem)

            @pl.loop(0, et // SG // 2)
            def _pair(i):
                sg0 = 2 * i
                _wait(rows, sem)
                _gather(sg0 + 1, rows1, sem1)
                _do(sg0, rows)
                _wait(rows1, sem1)
                _gather(sg0 + 2, rows, sem)
                _do(sg0 + 1, rows1)

            _wait(rows, sem)
            _do(et // SG - 1, rows)

        plsc.subcore_barrier()
        pltpu.sync_copy(acc.at[pl.ds(sub * 640, 640)],
                        outacc.at[cg, pl.ds(sub * 640, 640)])
        plsc.subcore_barrier()


def _phase_c(zflat, ext, rden, srcs, d2):
    return pl.kernel(
        _phase_c_body,
        out_type=jax.ShapeDtypeStruct((C8, NP, CW), _f32),
        mesh=_sc_mesh(),
        compiler_params=pltpu.CompilerParams(needs_layout_passes=False,
                                             use_tc_tiling_on_sc=False),
        scratch_types=[
            pltpu.VMEM((NP,), _f32),
            pltpu.VMEM((E // NS,), _f32),
            pltpu.VMEM((E // NS,), _f32),
            pltpu.VMEM((E // NS,), _i32),
            pltpu.VMEM((ECH,), _i32),
            pltpu.VMEM((E // NS // SG, SG), _i32),
            pltpu.VMEM((SG, CW), _f32),
            pltpu.VMEM((SG, CW), _f32),
            pltpu.VMEM((128, CW), _f32),
            pltpu.SemaphoreType.DMA,
            pltpu.SemaphoreType.DMA,
            pltpu.VMEM_SHARED((NP, CW), _f32),
        ],
    )(zflat, ext, rden, srcs, d2)


def _phase_d_body(acc_ref, bs_ref, out_ref):
    bsum = jnp.sum(bs_ref[...], axis=0)
    nq = OUT // CW
    for h in range(H):
        for q in range(nq):
            out_ref[:, h, q * CW:(q + 1) * CW] = (
                acc_ref[h * nq + q]
                + bsum[h, q * CW:(q + 1) * CW][None])


def _phase_d(outacc, bs):
    return pl.pallas_call(
        _phase_d_body,
        grid=(NB,),
        in_specs=[
            pl.BlockSpec((C8, BN, CW), lambda i: (0, i, 0)),
            pl.BlockSpec((R, H, OUT), lambda i: (0, 0, 0)),
        ],
        out_specs=pl.BlockSpec((BN, H, OUT), lambda i: (i, 0, 0)),
        out_shape=jax.ShapeDtypeStruct((N, H, OUT), _f32),
    )(outacc, bs)


def kernel(x, W0, al0, ar0, b0, W1, al1, ar1, b1, W2, al2, ar2, b2,
           ei0, ei1, ei2):
    Ws = jnp.stack([W0, W1, W2])
    als = jnp.stack([al0, al1, al2]).reshape(R, H, OUT)
    ars = jnp.stack([ar0, ar1, ar2]).reshape(R, H, OUT)
    bs = jnp.stack([b0, b1, b2])
    srcs = jnp.stack([ei0[0], ei1[0], ei2[0]]).reshape(R * E)
    d2 = jnp.stack([ei0[1], ei1[1], ei2[1]]).reshape(R, NCH, NSG, SG)

    Z, eln, ern = _phase_a(x, Ws, als, ars)
    elt, ert = _phase_a2(eln, ern)
    zflat = Z.reshape(R * C8 * N, CW)
    denp, ext = _phase_b(elt, ert, srcs, d2)
    rden = _phase_b2(denp)
    outacc = _phase_c(zflat, ext, rden, srcs, d2)
    return _phase_d(outacc, bs)


# speedup vs baseline: 18.0846x; 1.0034x over previous
import jax
import jax.numpy as jnp
from jax import lax
from jax.experimental import pallas as pl
from jax.experimental.pallas import tpu as pltpu
from jax.experimental.pallas import tpu_sc as plsc

N = 10000
NP = 10240
E = 160000
IN = 256
H = 4
OUT = 256
R = 3
CW = 64
C8 = 16
BN = 400
NB = N // BN
ECH = 2000
NCH = E // ECH
SG = 80
NSG = ECH // SG
NC, NS, L = 2, 16, 16

_i32 = jnp.int32
_f32 = jnp.float32


def _splat(v):
    return jnp.full((L,), v, _i32)


def _iota16():
    return lax.broadcasted_iota(_i32, (L,), 0)


def _sc_mesh():
    return plsc.VectorSubcoreMesh(core_axis_name="c", subcore_axis_name="s",
                                  num_cores=NC, num_subcores=NS)


def _phase_a_body(x_ref, w_ref, al_ref, ar_ref, z_ref, el_ref, er_ref):
    z = jnp.dot(x_ref[...], w_ref[0], preferred_element_type=_f32)
    zr = z.reshape(BN, H, OUT)
    el_ref[0] = jnp.sum(zr * al_ref[0][None], axis=-1)
    er_ref[0] = jnp.sum(zr * ar_ref[0][None], axis=-1)
    for c in range(C8):
        z_ref[0, c] = z[:, c * CW:(c + 1) * CW]


def _phase_a(x, Ws, als, ars):
    return pl.pallas_call(
        _phase_a_body,
        grid=(R, NB),
        in_specs=[
            pl.BlockSpec((BN, IN), lambda r, i: (i, 0)),
            pl.BlockSpec((1, IN, H * OUT), lambda r, i: (r, 0, 0)),
            pl.BlockSpec((1, H, OUT), lambda r, i: (r, 0, 0)),
            pl.BlockSpec((1, H, OUT), lambda r, i: (r, 0, 0)),
        ],
        out_specs=[
            pl.BlockSpec((1, C8, BN, CW), lambda r, i: (r, 0, i, 0)),
            pl.BlockSpec((1, BN, H), lambda r, i: (r, i, 0)),
            pl.BlockSpec((1, BN, H), lambda r, i: (r, i, 0)),
        ],
        out_shape=[
            jax.ShapeDtypeStruct((R, C8, N, CW), _f32),
            jax.ShapeDtypeStruct((R, N, H), _f32),
            jax.ShapeDtypeStruct((R, N, H), _f32),
        ],
        compiler_params=pltpu.CompilerParams(
            dimension_semantics=("arbitrary", "arbitrary")),
    )(x, Ws, als, ars)


def _phase_a2_body(el_ref, er_ref, elt_ref, ert_ref):
    elt_ref[0] = el_ref[0].T
    ert_ref[0] = er_ref[0].T


def _phase_a2(eln, ern):
    return pl.pallas_call(
        _phase_a2_body,
        grid=(R,),
        in_specs=[
            pl.BlockSpec((1, N, H), lambda r: (r, 0, 0)),
            pl.BlockSpec((1, N, H), lambda r: (r, 0, 0)),
        ],
        out_specs=[
            pl.BlockSpec((1, H, N), lambda r: (r, 0, 0)),
            pl.BlockSpec((1, H, N), lambda r: (r, 0, 0)),
        ],
        out_shape=[
            jax.ShapeDtypeStruct((R, H, N), _f32),
            jax.ShapeDtypeStruct((R, H, N), _f32),
        ],
    )(eln, ern)


def _phase_b_body(elt, ert, srcs, d2, denp, ext,
                  elbuf, erbuf, sbuf, d2buf, exb0, exb1, exb2, exb3,
                  rowbuf, zbuf, mbuf, dacc):
    core = lax.axis_index("c")
    sub = lax.axis_index("s")
    iota = _iota16()

    @pl.loop(0, NP // NS, unroll=8)
    def _(i):
        zbuf[i, :] = jnp.zeros((L,), _f32)

    pltpu.sync_copy(zbuf, dacc.at[pl.ds(sub * (NP // NS), NP // NS)])
    plsc.subcore_barrier()

    @pl.when(sub < 12)
    def _work():
        w = sub * NC + core
        r = w // 8
        eighth = w % 8
        pltpu.sync_copy(elt.at[r], elbuf)
        pltpu.sync_copy(ert.at[r], erbuf)

        @pl.loop(0, SG)
        def _(g):
            rowbuf[g, :] = jnp.zeros((L,), _f32)

        iota16 = _iota16()

        def _allmax(m):
            for s in (1, 2, 4, 8):
                mbuf[pl.ds(0, L)] = m
                v = plsc.load_gather(
                    mbuf, [jnp.bitwise_xor(iota16, _splat(s))])
                m = jnp.maximum(m, v)
            return m

        cvec = []
        for h in range(H):
            def _mx(buf):
                def body(g, m):
                    return jnp.maximum(m, buf[h, pl.ds(g * L, L)])
                m = lax.fori_loop(0, N // L, body,
                                  jnp.full((L,), -3.4e38, _f32))
                return _allmax(m)
            ch = _mx(elbuf) + _mx(erbuf)
            cvec.append(jnp.maximum(ch, 0.2 * ch))

        @pl.loop(0, 10)
        def _chunk(k):
            cidx = eighth * 10 + k
            pltpu.sync_copy(srcs.at[pl.ds(r * E + cidx * ECH, ECH)], sbuf)
            pltpu.sync_copy(d2.at[r, cidx], d2buf)

            @pl.loop(0, ECH // L, unroll=2)
            def _g16(g):
                s = sbuf[pl.ds(g * L, L)]
                d = d2buf[g // 5, pl.ds((g % 5) * L, L)]
                exbufs = [exb0, exb1, exb2, exb3]
                for h in range(H):
                    elg = plsc.load_gather(elbuf, [_splat(h), s])
                    erg = plsc.load_gather(erbuf, [_splat(h), d])
                    t = elg + erg
                    e = jnp.maximum(t, 0.2 * t)
                    exbufs[h][pl.ds(g * L, L)] = jnp.exp(e - cvec[h])

            for h, exb in enumerate((exb0, exb1, exb2, exb3)):
                pltpu.sync_copy(
                    exb, ext.at[pl.ds((r * H + h) * E + cidx * ECH, ECH)])

            @pl.loop(0, NSG)
            def _sg(sg):
                for h, exb in enumerate((exb0, exb1, exb2, exb3)):
                    for t in range(SG // L):
                        exv = exb[pl.ds(sg * SG + t * L, L)]
                        gi = iota + t * L
                        ci = _splat(r * H + h)
                        plsc.store_scatter(rowbuf, [gi, ci], exv)
                pltpu.sync_copy(rowbuf, dacc.at[d2buf.at[sg]], add=True)

    plsc.subcore_barrier()

    @pl.when(sub == 0)
    def _out():
        pltpu.sync_copy(dacc, denp.at[core])


def _phase_b(elt, ert, srcs, d2):
    return pl.kernel(
        _phase_b_body,
        out_type=[
            jax.ShapeDtypeStruct((NC, NP, 16), _f32),
            jax.ShapeDtypeStruct((R * H * E,), _f32),
        ],
        mesh=_sc_mesh(),
        compiler_params=pltpu.CompilerParams(needs_layout_passes=False,
                                             use_tc_tiling_on_sc=False),
        scratch_types=[
            pltpu.VMEM((H, N), _f32),
            pltpu.VMEM((H, N), _f32),
            pltpu.VMEM((ECH,), _i32),
            pltpu.VMEM((NSG, SG), _i32),
            pltpu.VMEM((ECH,), _f32),
            pltpu.VMEM((ECH,), _f32),
            pltpu.VMEM((ECH,), _f32),
            pltpu.VMEM((ECH,), _f32),
            pltpu.VMEM((SG, 16), _f32),
            pltpu.VMEM((NP // NS, 16), _f32),
            pltpu.VMEM((128,), _f32),
            pltpu.VMEM_SHARED((NP, 16), _f32),
        ],
    )(elt, ert, srcs, d2)


def _phase_b2_body(denp_ref, rden_ref):
    d = denp_ref[0] + denp_ref[1]
    rd = 1.0 / jnp.maximum(d, 1e-30)
    rden_ref[...] = rd.T.reshape(16 * NP)


def _phase_b2(denp):
    return pl.pallas_call(
        _phase_b2_body,
        grid=(1,),
        in_specs=[pl.BlockSpec((NC, NP, 16), lambda i: (0, 0, 0))],
        out_specs=pl.BlockSpec((16 * NP,), lambda i: (0,)),
        out_shape=jax.ShapeDtypeStruct((16 * NP,), _f32),
    )(denp)


def _phase_c_body(zflat, ext, rden, srcs, d2, outacc,
                  rdenbuf, exsl, albuf, gidx, sbuf, d2buf, rows, rows1,
                  zbuf, sem, sem1, sems, sems1, acc):
    core = lax.axis_index("c")
    sub = lax.axis_index("s")
    et = E // NS

    @pl.loop(0, 128, unroll=8)
    def _(i):
        for kk in range(CW // L):
            zbuf[i, pl.ds(kk * L, L)] = jnp.zeros((L,), _f32)

    @pl.loop(0, C8 // NC)
    def _chunk(cl):
        cg = core * (C8 // NC) + cl
        h = cg // (C8 // H)
        for j in range(5):
            pltpu.sync_copy(zbuf, acc.at[pl.ds(sub * 640 + j * 128, 128)])
        plsc.subcore_barrier()

        @pl.loop(0, R)
        def _rel(r):
            rh = r * H + h
            pltpu.sync_copy(rden.at[pl.ds(rh * NP, NP)], rdenbuf)
            pltpu.sync_copy(ext.at[pl.ds((r * H + h) * E + sub * et, et)],
                            exsl)
            off = (r * C8 + cg) * N
            for k in range(5):
                ci = sub * 5 + k
                pltpu.sync_copy(srcs.at[pl.ds(r * E + ci * ECH, ECH)], sbuf)
                pltpu.sync_copy(d2.at[r, ci], d2buf.at[pl.ds(k * NSG, NSG)])

                @pl.loop(0, ECH // L, unroll=8)
                def _off(g):
                    gidx[pl.ds(k * ECH + g * L, L)] = (
                        sbuf[pl.ds(g * L, L)] + off)

            @pl.loop(0, et // L, unroll=8)
            def _alpha(g):
                d = d2buf[g // 5, pl.ds((g % 5) * L, L)]
                rd = plsc.load_gather(rdenbuf, [d])
                albuf[pl.ds(g * L, L)] = exsl[pl.ds(g * L, L)] * rd

            def _gather(sg, buf, sm):
                return pltpu.async_copy(
                    zflat.at[gidx.at[pl.ds(sg * SG, SG)]], buf, sm)

            def _wait(buf, sm):
                pltpu.make_async_copy(
                    zflat.at[gidx.at[pl.ds(0, SG)]], buf, sm).wait()

            def _scale(sg, buf):
                @pl.loop(0, SG, unroll=4)
                def _s(g):
                    a = plsc.load_gather(albuf,
                                         [jnp.full((L,), sg * SG + g, _i32)])
                    for kk in range(CW // L):
                        buf[g, pl.ds(kk * L, L)] = (
                            buf[g, pl.ds(kk * L, L)] * a)

            def _scat(sg, buf, sm):
                return pltpu.async_copy(buf, acc.at[d2buf.at[sg]], sm,
                                        add=True)

            def _swait(buf, sm):
                pltpu.make_async_copy(buf, acc.at[d2buf.at[0]], sm).wait()

            _gather(0, rows, sem)
            _gather(1, rows1, sem1)
            _wait(rows, sem)
            _scale(0, rows)
            _scat(0, rows, sems)

            @pl.loop(0, (et // SG - 3) // 2)
            def _pair(i):
                _swait(rows, sems)
                _gather(2 * i + 2, rows, sem)
                _wait(rows1, sem1)
                _scale(2 * i + 1, rows1)
                _scat(2 * i + 1, rows1, sems1)
                _swait(rows1, sems1)
                _gather(2 * i + 3, rows1, sem1)
                _wait(rows, sem)
                _scale(2 * i + 2, rows)
                _scat(2 * i + 2, rows, sems)

            _wait(rows1, sem1)
            _scale(et // SG - 2, rows1)
            _scat(et // SG - 2, rows1, sems1)
            _swait(rows, sems)
            _gather(et // SG - 1, rows, sem)
            _wait(rows, sem)
            _scale(et // SG - 1, rows)
            _scat(et // SG - 1, rows, sems)
            _swait(rows, sems)
            _swait(rows1, sems1)

        plsc.subcore_barrier()
        pltpu.sync_copy(acc.at[pl.ds(sub * 640, 640)],
                        outacc.at[cg, pl.ds(sub * 640, 640)])
        plsc.subcore_barrier()


def _phase_c(zflat, ext, rden, srcs, d2):
    return pl.kernel(
        _phase_c_body,
        out_type=jax.ShapeDtypeStruct((C8, NP, CW), _f32),
        mesh=_sc_mesh(),
        compiler_params=pltpu.CompilerParams(needs_layout_passes=False,
                                             use_tc_tiling_on_sc=False),
        scratch_types=[
            pltpu.VMEM((NP,), _f32),
            pltpu.VMEM((E // NS,), _f32),
            pltpu.VMEM((E // NS,), _f32),
            pltpu.VMEM((E // NS,), _i32),
            pltpu.VMEM((ECH,), _i32),
            pltpu.VMEM((E // NS // SG, SG), _i32),
            pltpu.VMEM((SG, CW), _f32),
            pltpu.VMEM((SG, CW), _f32),
            pltpu.VMEM((128, CW), _f32),
            pltpu.SemaphoreType.DMA,
            pltpu.SemaphoreType.DMA,
            pltpu.SemaphoreType.DMA,
            pltpu.SemaphoreType.DMA,
            pltpu.VMEM_SHARED((NP, CW), _f32),
        ],
    )(zflat, ext, rden, srcs, d2)


def _phase_d_body(acc_ref, bs_ref, out_ref):
    bsum = jnp.sum(bs_ref[...], axis=0)
    nq = OUT // CW
    for h in range(H):
        for q in range(nq):
            out_ref[:, h, q * CW:(q + 1) * CW] = (
                acc_ref[h * nq + q]
                + bsum[h, q * CW:(q + 1) * CW][None])


def _phase_d(outacc, bs):
    return pl.pallas_call(
        _phase_d_body,
        grid=(NB,),
        in_specs=[
            pl.BlockSpec((C8, BN, CW), lambda i: (0, i, 0)),
            pl.BlockSpec((R, H, OUT), lambda i: (0, 0, 0)),
        ],
        out_specs=pl.BlockSpec((BN, H, OUT), lambda i: (i, 0, 0)),
        out_shape=jax.ShapeDtypeStruct((N, H, OUT), _f32),
    )(outacc, bs)


def kernel(x, W0, al0, ar0, b0, W1, al1, ar1, b1, W2, al2, ar2, b2,
           ei0, ei1, ei2):
    Ws = jnp.stack([W0, W1, W2])
    als = jnp.stack([al0, al1, al2]).reshape(R, H, OUT)
    ars = jnp.stack([ar0, ar1, ar2]).reshape(R, H, OUT)
    bs = jnp.stack([b0, b1, b2])
    srcs = jnp.stack([ei0[0], ei1[0], ei2[0]]).reshape(R * E)
    d2 = jnp.stack([ei0[1], ei1[1], ei2[1]]).reshape(R, NCH, NSG, SG)

    Z, eln, ern = _phase_a(x, Ws, als, ars)
    elt, ert = _phase_a2(eln, ern)
    zflat = Z.reshape(R * C8 * N, CW)
    denp, ext = _phase_b(elt, ert, srcs, d2)
    rden = _phase_b2(denp)
    outacc = _phase_c(zflat, ext, rden, srcs, d2)
    return _phase_d(outacc, bs)


# speedup vs baseline: 21.0171x; 1.1622x over previous
import jax
import jax.numpy as jnp
from jax import lax
from jax.experimental import pallas as pl
from jax.experimental.pallas import tpu as pltpu
from jax.experimental.pallas import tpu_sc as plsc

N = 10000
NP = 10240
E = 160000
IN = 256
H = 4
OUT = 256
R = 3
CW = 64
C8 = 16
BN = 400
NB = N // BN
ECH = 2000
NCH = E // ECH
SG = 80
NSG = ECH // SG
NC, NS, L = 2, 16, 16

_i32 = jnp.int32
_f32 = jnp.float32


def _splat(v):
    return jnp.full((L,), v, _i32)


def _iota16():
    return lax.broadcasted_iota(_i32, (L,), 0)


def _sc_mesh():
    return plsc.VectorSubcoreMesh(core_axis_name="c", subcore_axis_name="s",
                                  num_cores=NC, num_subcores=NS)


def _phase_a_body(x_ref, w_ref, al_ref, ar_ref, z_ref, el_ref, er_ref):
    z = jnp.dot(x_ref[...], w_ref[0], preferred_element_type=_f32)
    zr = z.reshape(BN, H, OUT)
    el_ref[0] = jnp.sum(zr * al_ref[0][None], axis=-1)
    er_ref[0] = jnp.sum(zr * ar_ref[0][None], axis=-1)
    for c in range(C8):
        z_ref[0, c] = z[:, c * CW:(c + 1) * CW]


def _phase_a(x, Ws, als, ars):
    return pl.pallas_call(
        _phase_a_body,
        grid=(R, NB),
        in_specs=[
            pl.BlockSpec((BN, IN), lambda r, i: (i, 0)),
            pl.BlockSpec((1, IN, H * OUT), lambda r, i: (r, 0, 0)),
            pl.BlockSpec((1, H, OUT), lambda r, i: (r, 0, 0)),
            pl.BlockSpec((1, H, OUT), lambda r, i: (r, 0, 0)),
        ],
        out_specs=[
            pl.BlockSpec((1, C8, BN, CW), lambda r, i: (r, 0, i, 0)),
            pl.BlockSpec((1, BN, H), lambda r, i: (r, i, 0)),
            pl.BlockSpec((1, BN, H), lambda r, i: (r, i, 0)),
        ],
        out_shape=[
            jax.ShapeDtypeStruct((R, C8, N, CW), _f32),
            jax.ShapeDtypeStruct((R, N, H), _f32),
            jax.ShapeDtypeStruct((R, N, H), _f32),
        ],
        compiler_params=pltpu.CompilerParams(
            dimension_semantics=("arbitrary", "arbitrary")),
    )(x, Ws, als, ars)


def _phase_a2_body(el_ref, er_ref, elt_ref, ert_ref):
    elt_ref[0] = el_ref[0].T
    ert_ref[0] = er_ref[0].T


def _phase_a2(eln, ern):
    return pl.pallas_call(
        _phase_a2_body,
        grid=(R,),
        in_specs=[
            pl.BlockSpec((1, N, H), lambda r: (r, 0, 0)),
            pl.BlockSpec((1, N, H), lambda r: (r, 0, 0)),
        ],
        out_specs=[
            pl.BlockSpec((1, H, N), lambda r: (r, 0, 0)),
            pl.BlockSpec((1, H, N), lambda r: (r, 0, 0)),
        ],
        out_shape=[
            jax.ShapeDtypeStruct((R, H, N), _f32),
            jax.ShapeDtypeStruct((R, H, N), _f32),
        ],
    )(eln, ern)


def _phase_b_body(elt, ert, srcs, d2, denp, ext,
                  elbuf, erbuf, sbuf, d2buf, exb0, exb1, exb2, exb3,
                  rowbuf, zbuf, mbuf, dacc):
    core = lax.axis_index("c")
    sub = lax.axis_index("s")
    iota = _iota16()

    @pl.loop(0, NP // NS, unroll=8)
    def _(i):
        zbuf[i, :] = jnp.zeros((L,), _f32)

    pltpu.sync_copy(zbuf, dacc.at[pl.ds(sub * (NP // NS), NP // NS)])
    plsc.subcore_barrier()

    @pl.when(sub < 12)
    def _work():
        w = sub * NC + core
        r = w // 8
        eighth = w % 8
        pltpu.sync_copy(elt.at[r], elbuf)
        pltpu.sync_copy(ert.at[r], erbuf)

        @pl.loop(0, SG)
        def _(g):
            rowbuf[g, :] = jnp.zeros((L,), _f32)

        iota16 = _iota16()

        def _allmax(m):
            for s in (1, 2, 4, 8):
                mbuf[pl.ds(0, L)] = m
                v = plsc.load_gather(
                    mbuf, [jnp.bitwise_xor(iota16, _splat(s))])
                m = jnp.maximum(m, v)
            return m

        cvec = []
        for h in range(H):
            def _mx(buf):
                def body(g, m):
                    return jnp.maximum(m, buf[h, pl.ds(g * L, L)])
                m = lax.fori_loop(0, N // L, body,
                                  jnp.full((L,), -3.4e38, _f32))
                return _allmax(m)
            ch = _mx(elbuf) + _mx(erbuf)
            cvec.append(jnp.maximum(ch, 0.2 * ch))

        @pl.loop(0, 10)
        def _chunk(k):
            cidx = eighth * 10 + k
            pltpu.sync_copy(srcs.at[pl.ds(r * E + cidx * ECH, ECH)], sbuf)
            pltpu.sync_copy(d2.at[r, cidx], d2buf)

            @pl.loop(0, ECH // L, unroll=2)
            def _g16(g):
                s = sbuf[pl.ds(g * L, L)]
                d = d2buf[g // 5, pl.ds((g % 5) * L, L)]
                exbufs = [exb0, exb1, exb2, exb3]
                for h in range(H):
                    elg = plsc.load_gather(elbuf, [_splat(h), s])
                    erg = plsc.load_gather(erbuf, [_splat(h), d])
                    t = elg + erg
                    e = jnp.maximum(t, 0.2 * t)
                    exbufs[h][pl.ds(g * L, L)] = jnp.exp(e - cvec[h])

            for h, exb in enumerate((exb0, exb1, exb2, exb3)):
                pltpu.sync_copy(
                    exb, ext.at[pl.ds((r * H + h) * E + cidx * ECH, ECH)])

            @pl.loop(0, NSG)
            def _sg(sg):
                for h, exb in enumerate((exb0, exb1, exb2, exb3)):
                    for t in range(SG // L):
                        exv = exb[pl.ds(sg * SG + t * L, L)]
                        gi = iota + t * L
                        ci = _splat(r * H + h)
                        plsc.store_scatter(rowbuf, [gi, ci], exv)
                pltpu.sync_copy(rowbuf, dacc.at[d2buf.at[sg]], add=True)

    plsc.subcore_barrier()

    @pl.when(sub == 0)
    def _out():
        pltpu.sync_copy(dacc, denp.at[core])


def _phase_b(elt, ert, srcs, d2):
    return pl.kernel(
        _phase_b_body,
        out_type=[
            jax.ShapeDtypeStruct((NC, NP, 16), _f32),
            jax.ShapeDtypeStruct((R * H * E,), _f32),
        ],
        mesh=_sc_mesh(),
        compiler_params=pltpu.CompilerParams(needs_layout_passes=False,
                                             use_tc_tiling_on_sc=False),
        scratch_types=[
            pltpu.VMEM((H, N), _f32),
            pltpu.VMEM((H, N), _f32),
            pltpu.VMEM((ECH,), _i32),
            pltpu.VMEM((NSG, SG), _i32),
            pltpu.VMEM((ECH,), _f32),
            pltpu.VMEM((ECH,), _f32),
            pltpu.VMEM((ECH,), _f32),
            pltpu.VMEM((ECH,), _f32),
            pltpu.VMEM((SG, 16), _f32),
            pltpu.VMEM((NP // NS, 16), _f32),
            pltpu.VMEM((128,), _f32),
            pltpu.VMEM_SHARED((NP, 16), _f32),
        ],
    )(elt, ert, srcs, d2)


def _phase_b2_body(denp_ref, rden_ref):
    d = denp_ref[0] + denp_ref[1]
    rd = 1.0 / jnp.maximum(d, 1e-30)
    rden_ref[...] = rd.T.reshape(16 * NP)


def _phase_b2(denp):
    return pl.pallas_call(
        _phase_b2_body,
        grid=(1,),
        in_specs=[pl.BlockSpec((NC, NP, 16), lambda i: (0, 0, 0))],
        out_specs=pl.BlockSpec((16 * NP,), lambda i: (0,)),
        out_shape=jax.ShapeDtypeStruct((16 * NP,), _f32),
    )(denp)


def _phase_c_body(zflat, ext, rden, srcs, d2, outacc,
                  rdenbuf, exsl, albuf, gidx, sbuf, d2buf, rows, rows1,
                  zbuf, sem, sem1, sems, sems1, acc):
    core = lax.axis_index("c")
    sub = lax.axis_index("s")
    et = E // NS

    @pl.loop(0, 128, unroll=8)
    def _(i):
        for kk in range(CW // L):
            zbuf[i, pl.ds(kk * L, L)] = jnp.zeros((L,), _f32)

    @pl.loop(0, C8 // NC)
    def _chunk(cl):
        cg = core * (C8 // NC) + cl
        h = cg // (C8 // H)
        for j in range(5):
            pltpu.sync_copy(zbuf, acc.at[pl.ds(sub * 640 + j * 128, 128)])
        plsc.subcore_barrier()

        @pl.loop(0, R)
        def _rel(r):
            rh = r * H + h
            pltpu.sync_copy(rden.at[pl.ds(rh * NP, NP)], rdenbuf)
            pltpu.sync_copy(ext.at[pl.ds((r * H + h) * E + sub * et, et)],
                            exsl)
            off = (r * C8 + cg) * N
            for k in range(5):
                ci = sub * 5 + k
                pltpu.sync_copy(srcs.at[pl.ds(r * E + ci * ECH, ECH)], sbuf)
                pltpu.sync_copy(d2.at[r, ci], d2buf.at[pl.ds(k * NSG, NSG)])

                @pl.loop(0, ECH // L, unroll=8)
                def _off(g):
                    gidx[pl.ds(k * ECH + g * L, L)] = (
                        sbuf[pl.ds(g * L, L)] + off)

            @pl.loop(0, et // L, unroll=8)
            def _alpha(g):
                d = d2buf[g // 5, pl.ds((g % 5) * L, L)]
                rd = plsc.load_gather(rdenbuf, [d])
                albuf[pl.ds(g * L, L)] = exsl[pl.ds(g * L, L)] * rd

            def _gather(sg, buf, sm):
                return pltpu.async_copy(
                    zflat.at[gidx.at[pl.ds(sg * SG, SG)]], buf, sm)

            def _wait(buf, sm):
                pltpu.make_async_copy(
                    zflat.at[gidx.at[pl.ds(0, SG)]], buf, sm).wait()

            def _scale(sg, buf):
                @plsc.parallel_loop(0, SG, unroll=8)
                def _s(g):
                    a = plsc.load_gather(albuf,
                                         [jnp.full((L,), sg * SG + g, _i32)])
                    for kk in range(CW // L):
                        buf[g, pl.ds(kk * L, L)] = (
                            buf[g, pl.ds(kk * L, L)] * a)

            def _scat(sg, buf, sm):
                return pltpu.async_copy(buf, acc.at[d2buf.at[sg]], sm,
                                        add=True)

            def _swait(buf, sm):
                pltpu.make_async_copy(buf, acc.at[d2buf.at[0]], sm).wait()

            _gather(0, rows, sem)
            _gather(1, rows1, sem1)
            _wait(rows, sem)
            _scale(0, rows)
            _scat(0, rows, sems)

            @pl.loop(0, (et // SG - 3) // 2)
            def _pair(i):
                _swait(rows, sems)
                _gather(2 * i + 2, rows, sem)
                _wait(rows1, sem1)
                _scale(2 * i + 1, rows1)
                _scat(2 * i + 1, rows1, sems1)
                _swait(rows1, sems1)
                _gather(2 * i + 3, rows1, sem1)
                _wait(rows, sem)
                _scale(2 * i + 2, rows)
                _scat(2 * i + 2, rows, sems)

            _wait(rows1, sem1)
            _scale(et // SG - 2, rows1)
            _scat(et // SG - 2, rows1, sems1)
            _swait(rows, sems)
            _gather(et // SG - 1, rows, sem)
            _wait(rows, sem)
            _scale(et // SG - 1, rows)
            _scat(et // SG - 1, rows, sems)
            _swait(rows, sems)
            _swait(rows1, sems1)

        plsc.subcore_barrier()
        pltpu.sync_copy(acc.at[pl.ds(sub * 640, 640)],
                        outacc.at[cg, pl.ds(sub * 640, 640)])
        plsc.subcore_barrier()


def _phase_c(zflat, ext, rden, srcs, d2):
    return pl.kernel(
        _phase_c_body,
        out_type=jax.ShapeDtypeStruct((C8, NP, CW), _f32),
        mesh=_sc_mesh(),
        compiler_params=pltpu.CompilerParams(needs_layout_passes=False,
                                             use_tc_tiling_on_sc=False),
        scratch_types=[
            pltpu.VMEM((NP,), _f32),
            pltpu.VMEM((E // NS,), _f32),
            pltpu.VMEM((E // NS,), _f32),
            pltpu.VMEM((E // NS,), _i32),
            pltpu.VMEM((ECH,), _i32),
            pltpu.VMEM((E // NS // SG, SG), _i32),
            pltpu.VMEM((SG, CW), _f32),
            pltpu.VMEM((SG, CW), _f32),
            pltpu.VMEM((128, CW), _f32),
            pltpu.SemaphoreType.DMA,
            pltpu.SemaphoreType.DMA,
            pltpu.SemaphoreType.DMA,
            pltpu.SemaphoreType.DMA,
            pltpu.VMEM_SHARED((NP, CW), _f32),
        ],
    )(zflat, ext, rden, srcs, d2)


def _phase_d_body(acc_ref, bs_ref, out_ref):
    bsum = jnp.sum(bs_ref[...], axis=0)
    nq = OUT // CW
    for h in range(H):
        for q in range(nq):
            out_ref[:, h, q * CW:(q + 1) * CW] = (
                acc_ref[h * nq + q]
                + bsum[h, q * CW:(q + 1) * CW][None])


def _phase_d(outacc, bs):
    return pl.pallas_call(
        _phase_d_body,
        grid=(NB,),
        in_specs=[
            pl.BlockSpec((C8, BN, CW), lambda i: (0, i, 0)),
            pl.BlockSpec((R, H, OUT), lambda i: (0, 0, 0)),
        ],
        out_specs=pl.BlockSpec((BN, H, OUT), lambda i: (i, 0, 0)),
        out_shape=jax.ShapeDtypeStruct((N, H, OUT), _f32),
    )(outacc, bs)


def kernel(x, W0, al0, ar0, b0, W1, al1, ar1, b1, W2, al2, ar2, b2,
           ei0, ei1, ei2):
    Ws = jnp.stack([W0, W1, W2])
    als = jnp.stack([al0, al1, al2]).reshape(R, H, OUT)
    ars = jnp.stack([ar0, ar1, ar2]).reshape(R, H, OUT)
    bs = jnp.stack([b0, b1, b2])
    srcs = jnp.stack([ei0[0], ei1[0], ei2[0]]).reshape(R * E)
    d2 = jnp.stack([ei0[1], ei1[1], ei2[1]]).reshape(R, NCH, NSG, SG)

    Z, eln, ern = _phase_a(x, Ws, als, ars)
    elt, ert = _phase_a2(eln, ern)
    zflat = Z.reshape(R * C8 * N, CW)
    denp, ext = _phase_b(elt, ert, srcs, d2)
    rden = _phase_b2(denp)
    outacc = _phase_c(zflat, ext, rden, srcs, d2)
    return _phase_d(outacc, bs)
